# baseline (speedup 1.0000x reference)
import functools

import jax
import jax.numpy as jnp
from jax import lax
from jax.experimental import pallas as pl
from jax.experimental.pallas import tpu as pltpu
from jax.experimental.pallas import tpu_sc as plsc

N = 50000
E = 800000
IN = 64
H = 128

NC = 2
NS = 16
K = 128
CW = 32

EE = E + N
NB_AG = 416
PIECE = 16
NPIECE = NB_AG // PIECE
EPAD = NS * NB_AG * K
NB_DG = 208
DUMP = N
NROW = N + 8

STRIPE = 3136
STRIPE_LAST = N - 15 * STRIPE
ZP = 112

_mesh = plsc.VectorSubcoreMesh(
    core_axis_name="c", subcore_axis_name="s", num_cores=NC, num_subcores=NS)

_sc_params = pltpu.CompilerParams(use_tc_tiling_on_sc=False)

f32 = jnp.float32


def _fill(ref, n, value):
    v = jnp.full((16,), value, dtype=f32)

    def body(i, _):
        ref[pl.ds(i * 16, 16)] = v
        return 0

    lax.fori_loop(0, n // 16, body, 0)


def _stripe(sid):
    return STRIPE * sid


@functools.partial(
    pl.kernel,
    out_type=jax.ShapeDtypeStruct((NC * N,), f32),
    mesh=_mesh,
    compiler_params=_sc_params,
    scratch_types=[
        pltpu.VMEM_SHARED((NROW,), f32),
        pltpu.VMEM((NB_DG, K), jnp.int32),
        pltpu.VMEM((K,), f32),
        pltpu.VMEM((STRIPE,), f32),
    ],
)
def _sc_deg(dst_hbm, out_hbm, deg_sp, idxb, onesv, zbuf):
    cid = lax.axis_index("c")
    sid = lax.axis_index("s")
    _fill(onesv, K, 1.0)
    _fill(zbuf, STRIPE, 0.0)
    off = _stripe(sid)

    @pl.when(sid < NS - 1)
    def _():
        pltpu.sync_copy(zbuf, deg_sp.at[pl.ds(off, STRIPE)])

    @pl.when(sid == NS - 1)
    def _():
        pltpu.sync_copy(zbuf.at[pl.ds(0, STRIPE_LAST)],
                        deg_sp.at[pl.ds(off, STRIPE_LAST)])
        pltpu.sync_copy(zbuf.at[pl.ds(0, 8)], deg_sp.at[pl.ds(N, 8)])

    plsc.subcore_barrier()

    wid = cid * NS + sid
    pltpu.sync_copy(dst_hbm.at[wid], idxb)

    def body(i, _):
        pltpu.sync_copy(onesv, deg_sp.at[idxb.at[i]], add=True)
        return 0

    lax.fori_loop(0, NB_DG, body, 0)
    plsc.subcore_barrier()

    @pl.when(sid < NS - 1)
    def _():
        pltpu.sync_copy(deg_sp.at[pl.ds(off, STRIPE)], zbuf)
        pltpu.sync_copy(zbuf, out_hbm.at[pl.ds(cid * N + off, STRIPE)])

    @pl.when(sid == NS - 1)
    def _():
        pltpu.sync_copy(deg_sp.at[pl.ds(off, STRIPE_LAST)],
                        zbuf.at[pl.ds(0, STRIPE_LAST)])
        pltpu.sync_copy(zbuf.at[pl.ds(0, STRIPE_LAST)],
                        out_hbm.at[pl.ds(cid * N + off, STRIPE_LAST)])


def _agg_impl(cn, tables, src_hbm, dst_hbm, out_hbm,
              agg_sp, srcv, dstv, rows, zbuf, sg, ss, semi):
    cid = lax.axis_index("c")
    sid = lax.axis_index("s")
    off = _stripe(sid)

    def zfill(i, _):
        zbuf[pl.ds(i * 16, 16), :] = jnp.zeros((16, CW), dtype=f32)
        return 0

    def xform(slot, jj):
        def t(i, _):
            r = lax.div(i, K // 16)
            c = lax.rem(i, K // 16)
            v = srcv[slot, r, pl.ds(c * 16, 16)]
            srcv[slot, r, pl.ds(c * 16, 16)] = v * cn + jj
            return 0
        lax.fori_loop(0, PIECE * (K // 16), t, 0)

    def process(jj):
        hs_hbm = tables[jj] if len(tables) > 1 else tables[0]
        needs_xform = len(tables) == 1
        lax.fori_loop(0, ZP // 16, zfill, 0)

        @pl.when(sid < NS - 1)
        def _():
            def z(p, _):
                pltpu.sync_copy(zbuf, agg_sp.at[pl.ds(off + ZP * p, ZP)])
                return 0
            lax.fori_loop(0, STRIPE // ZP, z, 0)

        @pl.when(sid == NS - 1)
        def _():
            def z(p, _):
                pltpu.sync_copy(zbuf, agg_sp.at[pl.ds(off + ZP * p, ZP)])
                return 0
            lax.fori_loop(0, STRIPE_LAST // ZP, z, 0)
            rem = STRIPE_LAST - (STRIPE_LAST // ZP) * ZP
            pltpu.sync_copy(zbuf.at[pl.ds(0, rem)],
                            agg_sp.at[pl.ds(off + STRIPE_LAST - rem, rem)])
            pltpu.sync_copy(zbuf.at[pl.ds(0, 8)], agg_sp.at[pl.ds(N, 8)])

        plsc.subcore_barrier()

        pltpu.sync_copy(src_hbm.at[sid, pl.ds(0, PIECE)], srcv.at[0])
        pltpu.sync_copy(dst_hbm.at[sid, pl.ds(0, PIECE)], dstv.at[0])
        if needs_xform:
            xform(0, jj)
        pltpu.async_copy(hs_hbm.at[srcv.at[0, 0]], rows.at[0], sg.at[0])
        pltpu.async_copy(hs_hbm.at[srcv.at[0, 1]], rows.at[1], sg.at[1])

        def group(g, _):
            for u in range(4):
                b = 4 * g + u
                u2 = (u + 2) % 4
                if u == 2:
                    q = lax.div(g, 4)
                    qn = q + 1
                    sl = lax.rem(qn, 2)

                    @pl.when((lax.rem(g, 4) == 0) & (qn < NPIECE))
                    def _():
                        nxt = pl.ds(qn * PIECE, PIECE)
                        pltpu.async_copy(src_hbm.at[sid, nxt],
                                         srcv.at[sl], semi)
                        pltpu.async_copy(dst_hbm.at[sid, nxt],
                                         dstv.at[sl], semi)

                    @pl.when((lax.rem(g, 4) == 3) & (qn < NPIECE))
                    def _():
                        dummy = pl.ds(0, PIECE)
                        pltpu.make_async_copy(src_hbm.at[sid, dummy],
                                              srcv.at[0], semi).wait()
                        pltpu.make_async_copy(dst_hbm.at[sid, dummy],
                                              dstv.at[0], semi).wait()
                        if needs_xform:
                            xform(sl, jj)

                @pl.when(b >= 2)
                def _():
                    pltpu.make_async_copy(
                        rows.at[u2], agg_sp.at[dstv.at[0, 0]],
                        ss.at[u2]).wait()

                @pl.when(b + 2 < NB_AG)
                def _():
                    bn = b + 2
                    pn = lax.div(bn, PIECE)
                    pltpu.async_copy(
                        hs_hbm.at[srcv.at[lax.rem(pn, 2),
                                         lax.rem(bn, PIECE)]],
                        rows.at[u2], sg.at[u2])

                pltpu.make_async_copy(
                    hs_hbm.at[srcv.at[0, 0]], rows.at[u], sg.at[u]).wait()
                p = lax.div(b, PIECE)
                pltpu.async_copy(
                    rows.at[u], agg_sp.at[dstv.at[lax.rem(p, 2),
                                                  lax.rem(b, PIECE)]],
                    ss.at[u], add=True)
            return 0

        lax.fori_loop(0, NB_AG // 4, group, 0)
        for u in (2, 3):
            pltpu.make_async_copy(
                rows.at[u], agg_sp.at[dstv.at[0, 0]], ss.at[u]).wait()
        plsc.subcore_barrier()

        @pl.when(sid < NS - 1)
        def _():
            def w(p_, _):
                o_ = off + ZP * p_
                pltpu.sync_copy(agg_sp.at[pl.ds(o_, ZP)], zbuf)
                pltpu.sync_copy(zbuf, out_hbm.at[pl.ds(o_, ZP), jj])
                return 0
            lax.fori_loop(0, STRIPE // ZP, w, 0)

        @pl.when(sid == NS - 1)
        def _():
            def w(p_, _):
                o_ = off + ZP * p_
                pltpu.sync_copy(agg_sp.at[pl.ds(o_, ZP)], zbuf)
                pltpu.sync_copy(zbuf, out_hbm.at[pl.ds(o_, ZP), jj])
                return 0
            lax.fori_loop(0, STRIPE_LAST // ZP, w, 0)
            rem = STRIPE_LAST - (STRIPE_LAST // ZP) * ZP
            o_ = off + STRIPE_LAST - rem
            pltpu.sync_copy(agg_sp.at[pl.ds(o_, rem)],
                            zbuf.at[pl.ds(0, rem)])
            pltpu.sync_copy(zbuf.at[pl.ds(0, rem)],
                            out_hbm.at[pl.ds(o_, rem), jj])

        plsc.subcore_barrier()

    npc = cn // 2

    @pl.when(cid == 0)
    def _():
        for t in range(npc):
            process(t)

    @pl.when(cid == 1)
    def _():
        for t in range(npc):
            process(npc + t)


def _agg_scratch():
    return [
        pltpu.VMEM_SHARED((NROW, CW), f32),
        pltpu.VMEM((2, PIECE, K), jnp.int32),
        pltpu.VMEM((2, PIECE, K), jnp.int32),
        pltpu.VMEM((4, K, CW), f32),
        pltpu.VMEM((ZP, CW), f32),
        pltpu.SemaphoreType.DMA((4,)),
        pltpu.SemaphoreType.DMA((4,)),
        pltpu.SemaphoreType.DMA,
    ]


@functools.partial(
    pl.kernel,
    out_type=jax.ShapeDtypeStruct((N, 2, CW), f32),
    mesh=_mesh,
    compiler_params=_sc_params,
    scratch_types=_agg_scratch(),
)
def _sc_agg2(h0, h1, src_hbm, dst_hbm, out_hbm, *scr):
    _agg_impl(2, [h0, h1], src_hbm, dst_hbm, out_hbm, *scr)


@functools.partial(
    pl.kernel,
    out_type=jax.ShapeDtypeStruct((N, 4, CW), f32),
    mesh=_mesh,
    compiler_params=_sc_params,
    scratch_types=_agg_scratch(),
)
def _sc_agg4(hs_hbm, src_hbm, dst_hbm, out_hbm, *scr):
    _agg_impl(4, [hs_hbm], src_hbm, dst_hbm, out_hbm, *scr)


RB = 2000
NRB = N // RB
EPS = 1e-5


def _deg_scale(degp_ref):
    deg = jnp.sum(degp_ref[...], axis=1, keepdims=True)
    return lax.rsqrt(deg)


def _tca_body(x_ref, degp_ref, o0, o1):
    xs = x_ref[...] * _deg_scale(degp_ref)
    o0[...] = xs[:, :CW]
    o1[...] = xs[:, CW:]


def _tc_a(x, degp):
    return pl.pallas_call(
        _tca_body,
        grid=(NRB,),
        in_specs=[
            pl.BlockSpec((RB, IN), lambda i: (i, 0)),
            pl.BlockSpec((RB, 2), lambda i: (i, 0)),
        ],
        out_specs=[pl.BlockSpec((RB, CW), lambda i: (i, 0))] * 2,
        out_shape=[jax.ShapeDtypeStruct((N, CW), f32)] * 2,
    )(x, degp)


def _relu1(a_ref, degp_ref, w1_ref, b1_ref):
    d = _deg_scale(degp_ref)
    h = jnp.dot(a_ref[...], w1_ref[...], preferred_element_type=f32)
    return jnp.maximum(h * d + b1_ref[...], 0.0), d


def _tcb1_body(a_ref, degp_ref, w1_ref, b1_ref, stats_ref):
    r, _ = _relu1(a_ref, degp_ref, w1_ref, b1_ref)
    stats_ref[0, 0, :] = jnp.sum(r, axis=0)
    stats_ref[0, 1, :] = jnp.sum(r * r, axis=0)


def _tc_b1(agg1, degp, W1, b1):
    return pl.pallas_call(
        _tcb1_body,
        grid=(NRB,),
        in_specs=[
            pl.BlockSpec((RB, IN), lambda i: (i, 0)),
            pl.BlockSpec((RB, 2), lambda i: (i, 0)),
            pl.BlockSpec((IN, H), lambda i: (0, 0)),
            pl.BlockSpec((H,), lambda i: (0,)),
        ],
        out_specs=pl.BlockSpec((1, 2, H), lambda i: (i, 0, 0)),
        out_shape=jax.ShapeDtypeStruct((NRB, 2, H), f32),
    )(agg1, degp, W1, b1)


def _tcb2_body(a_ref, degp_ref, w1_ref, b1_ref, stats_ref, g_ref, be_ref,
               o_ref):
    r, d = _relu1(a_ref, degp_ref, w1_ref, b1_ref)
    s = stats_ref[...]
    mean = jnp.sum(s[:, 0], axis=0) * (1.0 / N)
    var = jnp.sum(s[:, 1], axis=0) * (1.0 / N) - mean * mean
    inv = lax.rsqrt(var + EPS)
    y = (r - mean) * (inv * g_ref[...]) + be_ref[...]
    o_ref[...] = jnp.reshape(y * d, (RB * H,))


def _tc_b2(agg1, degp, W1, b1, stats, gamma, beta):
    return pl.pallas_call(
        _tcb2_body,
        grid=(NRB,),
        in_specs=[
            pl.BlockSpec((RB, IN), lambda i: (i, 0)),
            pl.BlockSpec((RB, 2), lambda i: (i, 0)),
            pl.BlockSpec((IN, H), lambda i: (0, 0)),
            pl.BlockSpec((H,), lambda i: (0,)),
            pl.BlockSpec((NRB, 2, H), lambda i: (0, 0, 0)),
            pl.BlockSpec((H,), lambda i: (0,)),
            pl.BlockSpec((H,), lambda i: (0,)),
        ],
        out_specs=pl.BlockSpec((RB * H,), lambda i: (i,)),
        out_shape=jax.ShapeDtypeStruct((N * H,), f32),
    )(agg1, degp, W1, b1, stats, gamma, beta)


def _tcc_body(a_ref, degp_ref, w2_ref, b2_ref, out_ref):
    d = _deg_scale(degp_ref)
    h = jnp.dot(a_ref[...], w2_ref[...], preferred_element_type=f32)
    out_ref[...] = jnp.maximum(h * d + b2_ref[...], 0.0)


def _tc_c(agg2, degp, W2, b2):
    return pl.pallas_call(
        _tcc_body,
        grid=(NRB,),
        in_specs=[
            pl.BlockSpec((RB, H), lambda i: (i, 0)),
            pl.BlockSpec((RB, 2), lambda i: (i, 0)),
            pl.BlockSpec((H, H), lambda i: (0, 0)),
            pl.BlockSpec((H,), lambda i: (0,)),
        ],
        out_specs=pl.BlockSpec((RB, H), lambda i: (i, 0)),
        out_shape=jax.ShapeDtypeStruct((N, H), f32),
    )(agg2, degp, W2, b2)


def kernel(x, edge_index, W1, b1, gamma, beta, W2, b2):
    src = edge_index[0]
    dst = edge_index[1]
    loop = jnp.arange(N, dtype=jnp.int32)
    npad = EPAD - EE
    src_p = jnp.concatenate([src, loop, jnp.zeros((npad,), jnp.int32)])
    dst_p = jnp.concatenate([dst, loop, jnp.full((npad,), DUMP, jnp.int32)])
    src_ag = src_p.reshape(NS, NB_AG, K)
    dst_ag = dst_p.reshape(NS, NB_AG, K)
    dst_dg = dst_p.reshape(NC * NS, NB_DG, K)

    degp = _sc_deg(dst_dg).reshape(NC, N).T
    xs1 = _tc_a(x, degp)
    agg1 = _sc_agg2(*xs1, src_ag, dst_ag)
    agg1f = agg1.reshape(N, IN)
    stats = _tc_b1(agg1f, degp, W1, b1)
    xs2 = _tc_b2(agg1f, degp, W1, b1, stats, gamma, beta)
    agg2 = _sc_agg4(xs2.reshape(4 * N, CW), src_ag, dst_ag)
    return _tc_c(agg2.reshape(N, H), degp, W2, b2)


# speedup vs baseline: 1.1197x; 1.1197x over previous
import functools

import jax
import jax.numpy as jnp
from jax import lax
from jax.experimental import pallas as pl
from jax.experimental.pallas import tpu as pltpu
from jax.experimental.pallas import tpu_sc as plsc

N = 50000
E = 800000
IN = 64
H = 128

NC = 2
NS = 16
K = 128
CW = 32

EE = E + N
NB_AG = 416
PIECE = 16
NPIECE = NB_AG // PIECE
EPAD = NS * NB_AG * K
NB_DG = 208
DUMP = N
NROW = N + 8

STRIPE = 3136
STRIPE_LAST = N - 15 * STRIPE
ZP = 112

_mesh = plsc.VectorSubcoreMesh(
    core_axis_name="c", subcore_axis_name="s", num_cores=NC, num_subcores=NS)

_sc_params = pltpu.CompilerParams(use_tc_tiling_on_sc=False)

f32 = jnp.float32


def _fill(ref, n, value):
    v = jnp.full((16,), value, dtype=f32)

    def body(i, _):
        ref[pl.ds(i * 16, 16)] = v
        return 0

    lax.fori_loop(0, n // 16, body, 0)


def _stripe(sid):
    return STRIPE * sid


@functools.partial(
    pl.kernel,
    out_type=jax.ShapeDtypeStruct((NC * N,), f32),
    mesh=_mesh,
    compiler_params=_sc_params,
    scratch_types=[
        pltpu.VMEM_SHARED((NROW,), f32),
        pltpu.VMEM((NB_DG, K), jnp.int32),
        pltpu.VMEM((K,), f32),
        pltpu.VMEM((STRIPE,), f32),
    ],
)
def _sc_deg(dst_hbm, out_hbm, deg_sp, idxb, onesv, zbuf):
    cid = lax.axis_index("c")
    sid = lax.axis_index("s")
    _fill(onesv, K, 1.0)
    _fill(zbuf, STRIPE, 0.0)
    off = _stripe(sid)

    @pl.when(sid < NS - 1)
    def _():
        pltpu.sync_copy(zbuf, deg_sp.at[pl.ds(off, STRIPE)])

    @pl.when(sid == NS - 1)
    def _():
        pltpu.sync_copy(zbuf.at[pl.ds(0, STRIPE_LAST)],
                        deg_sp.at[pl.ds(off, STRIPE_LAST)])
        pltpu.sync_copy(zbuf.at[pl.ds(0, 8)], deg_sp.at[pl.ds(N, 8)])

    plsc.subcore_barrier()

    wid = cid * NS + sid
    pltpu.sync_copy(dst_hbm.at[wid], idxb)

    def body(i, _):
        pltpu.sync_copy(onesv, deg_sp.at[idxb.at[i]], add=True)
        return 0

    lax.fori_loop(0, NB_DG, body, 0)
    plsc.subcore_barrier()

    @pl.when(sid < NS - 1)
    def _():
        pltpu.sync_copy(deg_sp.at[pl.ds(off, STRIPE)], zbuf)
        pltpu.sync_copy(zbuf, out_hbm.at[pl.ds(cid * N + off, STRIPE)])

    @pl.when(sid == NS - 1)
    def _():
        pltpu.sync_copy(deg_sp.at[pl.ds(off, STRIPE_LAST)],
                        zbuf.at[pl.ds(0, STRIPE_LAST)])
        pltpu.sync_copy(zbuf.at[pl.ds(0, STRIPE_LAST)],
                        out_hbm.at[pl.ds(cid * N + off, STRIPE_LAST)])


def _agg_impl(cn, tables, src_hbm, dst_hbm, out_list,
              agg_sp, srcv, dstv, rows, zbuf, sg, ss, semi):
    cid = lax.axis_index("c")
    sid = lax.axis_index("s")
    off = _stripe(sid)

    def zfill(i, _):
        zbuf[pl.ds(i * 16, 16), :] = jnp.zeros((16, CW), dtype=f32)
        return 0

    def process(jj):
        hs_hbm = tables[jj]
        out_hbm = out_list[jj]
        lax.fori_loop(0, ZP // 16, zfill, 0)

        @pl.when(sid < NS - 1)
        def _():
            def z(p, _):
                pltpu.sync_copy(zbuf, agg_sp.at[pl.ds(off + ZP * p, ZP)])
                return 0
            lax.fori_loop(0, STRIPE // ZP, z, 0)

        @pl.when(sid == NS - 1)
        def _():
            def z(p, _):
                pltpu.sync_copy(zbuf, agg_sp.at[pl.ds(off + ZP * p, ZP)])
                return 0
            lax.fori_loop(0, STRIPE_LAST // ZP, z, 0)
            rem = STRIPE_LAST - (STRIPE_LAST // ZP) * ZP
            pltpu.sync_copy(zbuf.at[pl.ds(0, rem)],
                            agg_sp.at[pl.ds(off + STRIPE_LAST - rem, rem)])
            pltpu.sync_copy(zbuf.at[pl.ds(0, 8)], agg_sp.at[pl.ds(N, 8)])

        plsc.subcore_barrier()

        pltpu.sync_copy(src_hbm.at[sid, pl.ds(0, PIECE)], srcv.at[0])
        pltpu.sync_copy(dst_hbm.at[sid, pl.ds(0, PIECE)], dstv.at[0])
        pltpu.async_copy(hs_hbm.at[srcv.at[0, 0]], rows.at[0], sg.at[0])
        pltpu.async_copy(hs_hbm.at[srcv.at[0, 1]], rows.at[1], sg.at[1])

        def group(g, _):
            for u in range(4):
                b = 4 * g + u
                u2 = (u + 2) % 4
                if u == 2:
                    q = lax.div(g, 4)
                    qn = q + 1
                    sl = lax.rem(qn, 2)

                    @pl.when((lax.rem(g, 4) == 0) & (qn < NPIECE))
                    def _():
                        nxt = pl.ds(qn * PIECE, PIECE)
                        pltpu.async_copy(src_hbm.at[sid, nxt],
                                         srcv.at[sl], semi)
                        pltpu.async_copy(dst_hbm.at[sid, nxt],
                                         dstv.at[sl], semi)

                    @pl.when((lax.rem(g, 4) == 3) & (qn < NPIECE))
                    def _():
                        dummy = pl.ds(0, PIECE)
                        pltpu.make_async_copy(src_hbm.at[sid, dummy],
                                              srcv.at[0], semi).wait()
                        pltpu.make_async_copy(dst_hbm.at[sid, dummy],
                                              dstv.at[0], semi).wait()

                @pl.when(b >= 2)
                def _():
                    pltpu.make_async_copy(
                        rows.at[u2], agg_sp.at[dstv.at[0, 0]],
                        ss.at[u2]).wait()

                @pl.when(b + 2 < NB_AG)
                def _():
                    bn = b + 2
                    pn = lax.div(bn, PIECE)
                    pltpu.async_copy(
                        hs_hbm.at[srcv.at[lax.rem(pn, 2),
                                         lax.rem(bn, PIECE)]],
                        rows.at[u2], sg.at[u2])

                pltpu.make_async_copy(
                    hs_hbm.at[srcv.at[0, 0]], rows.at[u], sg.at[u]).wait()
                p = lax.div(b, PIECE)
                pltpu.async_copy(
                    rows.at[u], agg_sp.at[dstv.at[lax.rem(p, 2),
                                                  lax.rem(b, PIECE)]],
                    ss.at[u], add=True)
            return 0

        lax.fori_loop(0, NB_AG // 4, group, 0)
        for u in (2, 3):
            pltpu.make_async_copy(
                rows.at[u], agg_sp.at[dstv.at[0, 0]], ss.at[u]).wait()
        plsc.subcore_barrier()

        @pl.when(sid < NS - 1)
        def _():
            def w(p_, _):
                o_ = off + ZP * p_
                pltpu.sync_copy(agg_sp.at[pl.ds(o_, ZP)], zbuf)
                pltpu.sync_copy(zbuf, out_hbm.at[pl.ds(o_, ZP)])
                return 0
            lax.fori_loop(0, STRIPE // ZP, w, 0)

        @pl.when(sid == NS - 1)
        def _():
            def w(p_, _):
                o_ = off + ZP * p_
                pltpu.sync_copy(agg_sp.at[pl.ds(o_, ZP)], zbuf)
                pltpu.sync_copy(zbuf, out_hbm.at[pl.ds(o_, ZP)])
                return 0
            lax.fori_loop(0, STRIPE_LAST // ZP, w, 0)
            rem = STRIPE_LAST - (STRIPE_LAST // ZP) * ZP
            o_ = off + STRIPE_LAST - rem
            pltpu.sync_copy(agg_sp.at[pl.ds(o_, rem)],
                            zbuf.at[pl.ds(0, rem)])
            pltpu.sync_copy(zbuf.at[pl.ds(0, rem)],
                            out_hbm.at[pl.ds(o_, rem)])

        plsc.subcore_barrier()

    npc = cn // 2

    @pl.when(cid == 0)
    def _():
        for t in range(npc):
            process(t)

    @pl.when(cid == 1)
    def _():
        for t in range(npc):
            process(npc + t)


def _agg_scratch():
    return [
        pltpu.VMEM_SHARED((NROW, CW), f32),
        pltpu.VMEM((2, PIECE, K), jnp.int32),
        pltpu.VMEM((2, PIECE, K), jnp.int32),
        pltpu.VMEM((4, K, CW), f32),
        pltpu.VMEM((ZP, CW), f32),
        pltpu.SemaphoreType.DMA((4,)),
        pltpu.SemaphoreType.DMA((4,)),
        pltpu.SemaphoreType.DMA,
    ]


@functools.partial(
    pl.kernel,
    out_type=[jax.ShapeDtypeStruct((N, CW), f32) for _ in range(2)],
    mesh=_mesh,
    compiler_params=_sc_params,
    scratch_types=_agg_scratch(),
)
def _sc_agg2(h0, h1, src_hbm, dst_hbm, o0, o1, *scr):
    _agg_impl(2, [h0, h1], src_hbm, dst_hbm, [o0, o1], *scr)


@functools.partial(
    pl.kernel,
    out_type=[jax.ShapeDtypeStruct((N, CW), f32) for _ in range(4)],
    mesh=_mesh,
    compiler_params=_sc_params,
    scratch_types=_agg_scratch(),
)
def _sc_agg4(h0, h1, h2, h3, src_hbm, dst_hbm, o0, o1, o2, o3, *scr):
    _agg_impl(4, [h0, h1, h2, h3], src_hbm, dst_hbm, [o0, o1, o2, o3], *scr)


RB = 2000
NRB = N // RB
EPS = 1e-5


def _deg_scale(degp_ref):
    deg = jnp.sum(degp_ref[...], axis=1, keepdims=True)
    return lax.rsqrt(deg)


def _tca_body(x_ref, degp_ref, o0, o1):
    xs = x_ref[...] * _deg_scale(degp_ref)
    o0[...] = xs[:, :CW]
    o1[...] = xs[:, CW:]


def _tc_a(x, degp):
    return pl.pallas_call(
        _tca_body,
        grid=(NRB,),
        in_specs=[
            pl.BlockSpec((RB, IN), lambda i: (i, 0)),
            pl.BlockSpec((RB, 2), lambda i: (i, 0)),
        ],
        out_specs=[pl.BlockSpec((RB, CW), lambda i: (i, 0))] * 2,
        out_shape=[jax.ShapeDtypeStruct((N, CW), f32)] * 2,
    )(x, degp)


def _relu1(a0, a1, degp_ref, w1_ref, b1_ref):
    d = _deg_scale(degp_ref)
    ar = jnp.concatenate([a0[...], a1[...]], axis=1)
    h = jnp.dot(ar, w1_ref[...], preferred_element_type=f32)
    return jnp.maximum(h * d + b1_ref[...], 0.0), d


def _tcb1_body(a0, a1, degp_ref, w1_ref, b1_ref, stats_ref):
    r, _ = _relu1(a0, a1, degp_ref, w1_ref, b1_ref)
    stats_ref[0, 0, :] = jnp.sum(r, axis=0)
    stats_ref[0, 1, :] = jnp.sum(r * r, axis=0)


def _tc_b1(aggs, degp, W1, b1):
    return pl.pallas_call(
        _tcb1_body,
        grid=(NRB,),
        in_specs=[pl.BlockSpec((RB, CW), lambda i: (i, 0))] * 2 + [
            pl.BlockSpec((RB, 2), lambda i: (i, 0)),
            pl.BlockSpec((IN, H), lambda i: (0, 0)),
            pl.BlockSpec((H,), lambda i: (0,)),
        ],
        out_specs=pl.BlockSpec((1, 2, H), lambda i: (i, 0, 0)),
        out_shape=jax.ShapeDtypeStruct((NRB, 2, H), f32),
    )(*aggs, degp, W1, b1)


def _tcb2_body(a0, a1, degp_ref, w1_ref, b1_ref, stats_ref, g_ref, be_ref,
               o0, o1, o2, o3):
    r, d = _relu1(a0, a1, degp_ref, w1_ref, b1_ref)
    s = stats_ref[...]
    mean = jnp.sum(s[:, 0], axis=0) * (1.0 / N)
    var = jnp.sum(s[:, 1], axis=0) * (1.0 / N) - mean * mean
    inv = lax.rsqrt(var + EPS)
    y = (r - mean) * (inv * g_ref[...]) + be_ref[...]
    xs2 = y * d
    for j, o in enumerate((o0, o1, o2, o3)):
        o[...] = xs2[:, CW * j:CW * (j + 1)]


def _tc_b2(aggs, degp, W1, b1, stats, gamma, beta):
    return pl.pallas_call(
        _tcb2_body,
        grid=(NRB,),
        in_specs=[pl.BlockSpec((RB, CW), lambda i: (i, 0))] * 2 + [
            pl.BlockSpec((RB, 2), lambda i: (i, 0)),
            pl.BlockSpec((IN, H), lambda i: (0, 0)),
            pl.BlockSpec((H,), lambda i: (0,)),
            pl.BlockSpec((NRB, 2, H), lambda i: (0, 0, 0)),
            pl.BlockSpec((H,), lambda i: (0,)),
            pl.BlockSpec((H,), lambda i: (0,)),
        ],
        out_specs=[pl.BlockSpec((RB, CW), lambda i: (i, 0))] * 4,
        out_shape=[jax.ShapeDtypeStruct((N, CW), f32)] * 4,
    )(*aggs, degp, W1, b1, stats, gamma, beta)


def _tcc_body(a0, a1, a2, a3, degp_ref, w2_ref, b2_ref, out_ref):
    d = _deg_scale(degp_ref)
    ag = jnp.concatenate([a0[...], a1[...], a2[...], a3[...]], axis=1)
    h = jnp.dot(ag, w2_ref[...], preferred_element_type=f32)
    out_ref[...] = jnp.maximum(h * d + b2_ref[...], 0.0)


def _tc_c(aggs, degp, W2, b2):
    return pl.pallas_call(
        _tcc_body,
        grid=(NRB,),
        in_specs=[pl.BlockSpec((RB, CW), lambda i: (i, 0))] * 4 + [
            pl.BlockSpec((RB, 2), lambda i: (i, 0)),
            pl.BlockSpec((H, H), lambda i: (0, 0)),
            pl.BlockSpec((H,), lambda i: (0,)),
        ],
        out_specs=pl.BlockSpec((RB, H), lambda i: (i, 0)),
        out_shape=jax.ShapeDtypeStruct((N, H), f32),
    )(*aggs, degp, W2, b2)


def kernel(x, edge_index, W1, b1, gamma, beta, W2, b2):
    src = edge_index[0]
    dst = edge_index[1]
    loop = jnp.arange(N, dtype=jnp.int32)
    npad = EPAD - EE
    src_p = jnp.concatenate([src, loop, jnp.zeros((npad,), jnp.int32)])
    dst_p = jnp.concatenate([dst, loop, jnp.full((npad,), DUMP, jnp.int32)])
    src_ag = src_p.reshape(NS, NB_AG, K)
    dst_ag = dst_p.reshape(NS, NB_AG, K)
    dst_dg = dst_p.reshape(NC * NS, NB_DG, K)

    degp = _sc_deg(dst_dg).reshape(NC, N).T
    xs1 = _tc_a(x, degp)
    agg1 = _sc_agg2(*xs1, src_ag, dst_ag)
    stats = _tc_b1(agg1, degp, W1, b1)
    xs2 = _tc_b2(agg1, degp, W1, b1, stats, gamma, beta)
    agg2 = _sc_agg4(*xs2, src_ag, dst_ag)
    return _tc_c(agg2, degp, W2, b2)


# speedup vs baseline: 1.1498x; 1.0268x over previous
import functools

import jax
import jax.numpy as jnp
from jax import lax
from jax.experimental import pallas as pl
from jax.experimental.pallas import tpu as pltpu
from jax.experimental.pallas import tpu_sc as plsc

N = 50000
E = 800000
IN = 64
H = 128

NC = 2
NS = 16
K = 128
CW = 32

EE = E + N
NB_AG = 416
PIECE = 16
NPIECE = NB_AG // PIECE
EPAD = NS * NB_AG * K
NB_DG = 208
EB = E // K
MB15 = 15 * NB_AG
TBLK = 406
DUMP = N
NROW = N + 8

STRIPE = 3136
STRIPE_LAST = N - 15 * STRIPE
ZP = 112

_mesh = plsc.VectorSubcoreMesh(
    core_axis_name="c", subcore_axis_name="s", num_cores=NC, num_subcores=NS)

_sc_params = pltpu.CompilerParams(use_tc_tiling_on_sc=False)

f32 = jnp.float32


def _fill(ref, n, value):
    v = jnp.full((16,), value, dtype=f32)

    def body(i, _):
        ref[pl.ds(i * 16, 16)] = v
        return 0

    lax.fori_loop(0, n // 16, body, 0)


def _stripe(sid):
    return STRIPE * sid


@functools.partial(
    pl.kernel,
    out_type=jax.ShapeDtypeStruct((NC * N,), f32),
    mesh=_mesh,
    compiler_params=_sc_params,
    scratch_types=[
        pltpu.VMEM_SHARED((NROW,), f32),
        pltpu.VMEM((NB_DG, K), jnp.int32),
        pltpu.VMEM((K,), f32),
        pltpu.VMEM((STRIPE,), f32),
    ],
)
def _sc_deg(dst_hbm, tail_hbm, out_hbm, deg_sp, idxb, onesv, zbuf):
    cid = lax.axis_index("c")
    sid = lax.axis_index("s")
    _fill(onesv, K, 1.0)
    _fill(zbuf, STRIPE, 0.0)
    off = _stripe(sid)

    @pl.when(sid < NS - 1)
    def _():
        pltpu.sync_copy(zbuf, deg_sp.at[pl.ds(off, STRIPE)])

    @pl.when(sid == NS - 1)
    def _():
        pltpu.sync_copy(zbuf.at[pl.ds(0, STRIPE_LAST)],
                        deg_sp.at[pl.ds(off, STRIPE_LAST)])
        pltpu.sync_copy(zbuf.at[pl.ds(0, 8)], deg_sp.at[pl.ds(N, 8)])

    plsc.subcore_barrier()

    wid = cid * NS + sid

    @pl.when(wid < 30)
    def _():
        pltpu.sync_copy(dst_hbm.at[pl.ds(wid * NB_DG, NB_DG)], idxb)

    @pl.when(wid == 30)
    def _():
        pltpu.sync_copy(dst_hbm.at[pl.ds(MB15, EB - MB15)],
                        idxb.at[pl.ds(0, EB - MB15)])
        pltpu.sync_copy(tail_hbm.at[pl.ds(0, NB_DG - (EB - MB15))],
                        idxb.at[pl.ds(EB - MB15, NB_DG - (EB - MB15))])

    @pl.when(wid == 31)
    def _():
        pltpu.sync_copy(tail_hbm.at[pl.ds(TBLK - NB_DG, NB_DG)], idxb)

    def body(i, _):
        pltpu.sync_copy(onesv, deg_sp.at[idxb.at[i]], add=True)
        return 0

    lax.fori_loop(0, NB_DG, body, 0)
    plsc.subcore_barrier()

    @pl.when(sid < NS - 1)
    def _():
        pltpu.sync_copy(deg_sp.at[pl.ds(off, STRIPE)], zbuf)
        pltpu.sync_copy(zbuf, out_hbm.at[pl.ds(cid * N + off, STRIPE)])

    @pl.when(sid == NS - 1)
    def _():
        pltpu.sync_copy(deg_sp.at[pl.ds(off, STRIPE_LAST)],
                        zbuf.at[pl.ds(0, STRIPE_LAST)])
        pltpu.sync_copy(zbuf.at[pl.ds(0, STRIPE_LAST)],
                        out_hbm.at[pl.ds(cid * N + off, STRIPE_LAST)])


def _agg_impl(cn, tables, src_hbm, dst_hbm, tsrc_hbm, tdst_hbm, out_list,
              agg_sp, srcv, dstv, rows, zbuf, sg, ss, semi):
    cid = lax.axis_index("c")
    sid = lax.axis_index("s")
    off = _stripe(sid)

    def zfill(i, _):
        zbuf[pl.ds(i * 16, 16), :] = jnp.zeros((16, CW), dtype=f32)
        return 0

    def process(jj):
        hs_hbm = tables[jj]
        out_hbm = out_list[jj]
        lax.fori_loop(0, ZP // 16, zfill, 0)

        @pl.when(sid < NS - 1)
        def _():
            def z(p, _):
                pltpu.sync_copy(zbuf, agg_sp.at[pl.ds(off + ZP * p, ZP)])
                return 0
            lax.fori_loop(0, STRIPE // ZP, z, 0)

        @pl.when(sid == NS - 1)
        def _():
            def z(p, _):
                pltpu.sync_copy(zbuf, agg_sp.at[pl.ds(off + ZP * p, ZP)])
                return 0
            lax.fori_loop(0, STRIPE_LAST // ZP, z, 0)
            rem = STRIPE_LAST - (STRIPE_LAST // ZP) * ZP
            pltpu.sync_copy(zbuf.at[pl.ds(0, rem)],
                            agg_sp.at[pl.ds(off + STRIPE_LAST - rem, rem)])
            pltpu.sync_copy(zbuf.at[pl.ds(0, 8)], agg_sp.at[pl.ds(N, 8)])

        plsc.subcore_barrier()

        @pl.when(sid < NS - 1)
        def _():
            p0 = pl.ds(sid * NB_AG, PIECE)
            pltpu.sync_copy(src_hbm.at[p0], srcv.at[0])
            pltpu.sync_copy(dst_hbm.at[p0], dstv.at[0])

        @pl.when(sid == NS - 1)
        def _():
            m = EB - MB15
            pltpu.sync_copy(src_hbm.at[pl.ds(MB15, m)],
                            srcv.at[0, pl.ds(0, m)])
            pltpu.sync_copy(dst_hbm.at[pl.ds(MB15, m)],
                            dstv.at[0, pl.ds(0, m)])
            pltpu.sync_copy(tsrc_hbm.at[pl.ds(0, PIECE - m)],
                            srcv.at[0, pl.ds(m, PIECE - m)])
            pltpu.sync_copy(tdst_hbm.at[pl.ds(0, PIECE - m)],
                            dstv.at[0, pl.ds(m, PIECE - m)])
        pltpu.async_copy(hs_hbm.at[srcv.at[0, 0]], rows.at[0], sg.at[0])
        pltpu.async_copy(hs_hbm.at[srcv.at[0, 1]], rows.at[1], sg.at[1])

        def group(g, _):
            for u in range(4):
                b = 4 * g + u
                u2 = (u + 2) % 4
                if u == 2:
                    q = lax.div(g, 4)
                    qn = q + 1
                    sl = lax.rem(qn, 2)

                    @pl.when((lax.rem(g, 4) == 0) & (qn < NPIECE))
                    def _():
                        @pl.when(sid < NS - 1)
                        def _():
                            nxt = pl.ds(sid * NB_AG + qn * PIECE, PIECE)
                            pltpu.async_copy(src_hbm.at[nxt],
                                             srcv.at[sl], semi)
                            pltpu.async_copy(dst_hbm.at[nxt],
                                             dstv.at[sl], semi)

                        @pl.when(sid == NS - 1)
                        def _():
                            nxt = pl.ds(qn * PIECE - (EB - MB15), PIECE)
                            pltpu.async_copy(tsrc_hbm.at[nxt],
                                             srcv.at[sl], semi)
                            pltpu.async_copy(tdst_hbm.at[nxt],
                                             dstv.at[sl], semi)

                    @pl.when((lax.rem(g, 4) == 3) & (qn < NPIECE))
                    def _():
                        dummy = pl.ds(0, PIECE)
                        pltpu.make_async_copy(src_hbm.at[dummy],
                                              srcv.at[0], semi).wait()
                        pltpu.make_async_copy(dst_hbm.at[dummy],
                                              dstv.at[0], semi).wait()

                @pl.when(b >= 2)
                def _():
                    pltpu.make_async_copy(
                        rows.at[u2], agg_sp.at[dstv.at[0, 0]],
                        ss.at[u2]).wait()

                @pl.when(b + 2 < NB_AG)
                def _():
                    bn = b + 2
                    pn = lax.div(bn, PIECE)
                    pltpu.async_copy(
                        hs_hbm.at[srcv.at[lax.rem(pn, 2),
                                         lax.rem(bn, PIECE)]],
                        rows.at[u2], sg.at[u2])

                pltpu.make_async_copy(
                    hs_hbm.at[srcv.at[0, 0]], rows.at[u], sg.at[u]).wait()
                p = lax.div(b, PIECE)
                pltpu.async_copy(
                    rows.at[u], agg_sp.at[dstv.at[lax.rem(p, 2),
                                                  lax.rem(b, PIECE)]],
                    ss.at[u], add=True)
            return 0

        lax.fori_loop(0, NB_AG // 4, group, 0)
        for u in (2, 3):
            pltpu.make_async_copy(
                rows.at[u], agg_sp.at[dstv.at[0, 0]], ss.at[u]).wait()
        plsc.subcore_barrier()

        @pl.when(sid < NS - 1)
        def _():
            def w(p_, _):
                o_ = off + ZP * p_
                pltpu.sync_copy(agg_sp.at[pl.ds(o_, ZP)], zbuf)
                pltpu.sync_copy(zbuf, out_hbm.at[pl.ds(o_, ZP)])
                return 0
            lax.fori_loop(0, STRIPE // ZP, w, 0)

        @pl.when(sid == NS - 1)
        def _():
            def w(p_, _):
                o_ = off + ZP * p_
                pltpu.sync_copy(agg_sp.at[pl.ds(o_, ZP)], zbuf)
                pltpu.sync_copy(zbuf, out_hbm.at[pl.ds(o_, ZP)])
                return 0
            lax.fori_loop(0, STRIPE_LAST // ZP, w, 0)
            rem = STRIPE_LAST - (STRIPE_LAST // ZP) * ZP
            o_ = off + STRIPE_LAST - rem
            pltpu.sync_copy(agg_sp.at[pl.ds(o_, rem)],
                            zbuf.at[pl.ds(0, rem)])
            pltpu.sync_copy(zbuf.at[pl.ds(0, rem)],
                            out_hbm.at[pl.ds(o_, rem)])

        plsc.subcore_barrier()

    npc = cn // 2

    @pl.when(cid == 0)
    def _():
        for t in range(npc):
            process(t)

    @pl.when(cid == 1)
    def _():
        for t in range(npc):
            process(npc + t)


def _agg_scratch():
    return [
        pltpu.VMEM_SHARED((NROW, CW), f32),
        pltpu.VMEM((2, PIECE, K), jnp.int32),
        pltpu.VMEM((2, PIECE, K), jnp.int32),
        pltpu.VMEM((4, K, CW), f32),
        pltpu.VMEM((ZP, CW), f32),
        pltpu.SemaphoreType.DMA((4,)),
        pltpu.SemaphoreType.DMA((4,)),
        pltpu.SemaphoreType.DMA,
    ]


@functools.partial(
    pl.kernel,
    out_type=[jax.ShapeDtypeStruct((N, CW), f32) for _ in range(2)],
    mesh=_mesh,
    compiler_params=_sc_params,
    scratch_types=_agg_scratch(),
)
def _sc_agg2(h0, h1, src_hbm, dst_hbm, tsrc, tdst, o0, o1, *scr):
    _agg_impl(2, [h0, h1], src_hbm, dst_hbm, tsrc, tdst, [o0, o1], *scr)


@functools.partial(
    pl.kernel,
    out_type=[jax.ShapeDtypeStruct((N, CW), f32) for _ in range(4)],
    mesh=_mesh,
    compiler_params=_sc_params,
    scratch_types=_agg_scratch(),
)
def _sc_agg4(h0, h1, h2, h3, src_hbm, dst_hbm, tsrc, tdst,
             o0, o1, o2, o3, *scr):
    _agg_impl(4, [h0, h1, h2, h3], src_hbm, dst_hbm, tsrc, tdst,
              [o0, o1, o2, o3], *scr)


RB = 2000
NRB = N // RB
EPS = 1e-5


def _deg_scale(degp_ref):
    deg = jnp.sum(degp_ref[...], axis=1, keepdims=True)
    return lax.rsqrt(deg)


def _tca_body(x_ref, degp_ref, o0, o1):
    xs = x_ref[...] * _deg_scale(degp_ref)
    o0[...] = xs[:, :CW]
    o1[...] = xs[:, CW:]


def _tc_a(x, degp):
    return pl.pallas_call(
        _tca_body,
        grid=(NRB,),
        in_specs=[
            pl.BlockSpec((RB, IN), lambda i: (i, 0)),
            pl.BlockSpec((RB, 2), lambda i: (i, 0)),
        ],
        out_specs=[pl.BlockSpec((RB, CW), lambda i: (i, 0))] * 2,
        out_shape=[jax.ShapeDtypeStruct((N, CW), f32)] * 2,
    )(x, degp)


def _relu1(a0, a1, degp_ref, w1_ref, b1_ref):
    d = _deg_scale(degp_ref)
    ar = jnp.concatenate([a0[...], a1[...]], axis=1)
    h = jnp.dot(ar, w1_ref[...], preferred_element_type=f32)
    return jnp.maximum(h * d + b1_ref[...], 0.0), d


def _tcb1_body(a0, a1, degp_ref, w1_ref, b1_ref, stats_ref):
    r, _ = _relu1(a0, a1, degp_ref, w1_ref, b1_ref)
    stats_ref[0, 0, :] = jnp.sum(r, axis=0)
    stats_ref[0, 1, :] = jnp.sum(r * r, axis=0)


def _tc_b1(aggs, degp, W1, b1):
    return pl.pallas_call(
        _tcb1_body,
        grid=(NRB,),
        in_specs=[pl.BlockSpec((RB, CW), lambda i: (i, 0))] * 2 + [
            pl.BlockSpec((RB, 2), lambda i: (i, 0)),
            pl.BlockSpec((IN, H), lambda i: (0, 0)),
            pl.BlockSpec((H,), lambda i: (0,)),
        ],
        out_specs=pl.BlockSpec((1, 2, H), lambda i: (i, 0, 0)),
        out_shape=jax.ShapeDtypeStruct((NRB, 2, H), f32),
    )(*aggs, degp, W1, b1)


def _tcb2_body(a0, a1, degp_ref, w1_ref, b1_ref, stats_ref, g_ref, be_ref,
               o0, o1, o2, o3):
    r, d = _relu1(a0, a1, degp_ref, w1_ref, b1_ref)
    s = stats_ref[...]
    mean = jnp.sum(s[:, 0], axis=0) * (1.0 / N)
    var = jnp.sum(s[:, 1], axis=0) * (1.0 / N) - mean * mean
    inv = lax.rsqrt(var + EPS)
    y = (r - mean) * (inv * g_ref[...]) + be_ref[...]
    xs2 = y * d
    for j, o in enumerate((o0, o1, o2, o3)):
        o[...] = xs2[:, CW * j:CW * (j + 1)]


def _tc_b2(aggs, degp, W1, b1, stats, gamma, beta):
    return pl.pallas_call(
        _tcb2_body,
        grid=(NRB,),
        in_specs=[pl.BlockSpec((RB, CW), lambda i: (i, 0))] * 2 + [
            pl.BlockSpec((RB, 2), lambda i: (i, 0)),
            pl.BlockSpec((IN, H), lambda i: (0, 0)),
            pl.BlockSpec((H,), lambda i: (0,)),
            pl.BlockSpec((NRB, 2, H), lambda i: (0, 0, 0)),
            pl.BlockSpec((H,), lambda i: (0,)),
            pl.BlockSpec((H,), lambda i: (0,)),
        ],
        out_specs=[pl.BlockSpec((RB, CW), lambda i: (i, 0))] * 4,
        out_shape=[jax.ShapeDtypeStruct((N, CW), f32)] * 4,
    )(*aggs, degp, W1, b1, stats, gamma, beta)


def _tcc_body(a0, a1, a2, a3, degp_ref, w2_ref, b2_ref, out_ref):
    d = _deg_scale(degp_ref)
    ag = jnp.concatenate([a0[...], a1[...], a2[...], a3[...]], axis=1)
    h = jnp.dot(ag, w2_ref[...], preferred_element_type=f32)
    out_ref[...] = jnp.maximum(h * d + b2_ref[...], 0.0)


def _tc_c(aggs, degp, W2, b2):
    return pl.pallas_call(
        _tcc_body,
        grid=(NRB,),
        in_specs=[pl.BlockSpec((RB, CW), lambda i: (i, 0))] * 4 + [
            pl.BlockSpec((RB, 2), lambda i: (i, 0)),
            pl.BlockSpec((H, H), lambda i: (0, 0)),
            pl.BlockSpec((H,), lambda i: (0,)),
        ],
        out_specs=pl.BlockSpec((RB, H), lambda i: (i, 0)),
        out_shape=jax.ShapeDtypeStruct((N, H), f32),
    )(*aggs, degp, W2, b2)


def kernel(x, edge_index, W1, b1, gamma, beta, W2, b2):
    src2d = edge_index[0].reshape(EB, K)
    dst2d = edge_index[1].reshape(EB, K)
    loop = jnp.arange(N, dtype=jnp.int32)
    npad = EPAD - EE
    tsrc = jnp.concatenate(
        [loop, jnp.zeros((npad,), jnp.int32)]).reshape(TBLK, K)
    tdst = jnp.concatenate(
        [loop, jnp.full((npad,), DUMP, jnp.int32)]).reshape(TBLK, K)

    degp = _sc_deg(dst2d, tdst).reshape(NC, N).T
    xs1 = _tc_a(x, degp)
    agg1 = _sc_agg2(*xs1, src2d, dst2d, tsrc, tdst)
    stats = _tc_b1(agg1, degp, W1, b1)
    xs2 = _tc_b2(agg1, degp, W1, b1, stats, gamma, beta)
    agg2 = _sc_agg4(*xs2, src2d, dst2d, tsrc, tdst)
    return _tc_c(agg2, degp, W2, b2)
